# Initial kernel scaffold; baseline (speedup 1.0000x reference)
import jax
import jax.numpy as jnp
from jax.experimental import pallas as pl


def kernel(x, edge_index, edge_weight, batch, W0, b0, W1, b1, g0, be0, g1, be1, Wp0, bp0, Wp1, bp1, Wp2, bp2):
    raise NotImplementedError("write your pallas kernel here")


# speedup vs baseline: 2.7635x; 2.7635x over previous
import functools

import jax
import jax.numpy as jnp
from jax import lax
from jax.experimental import pallas as pl
from jax.experimental.pallas import tpu as pltpu
from jax.experimental.pallas import tpu_sc as plsc

_N = 10000
_E = 160000
_D = 256
_HD = 128
_DO = 128
_G = 64
_RB = 400
_NRB = _N // _RB
_EB = 80
_NSUB = 16
_ROWS_PER_SUB = 624
_ROWS_TAIL = _N - _NSUB * _ROWS_PER_SUB
_EDGES_PER_SUB = _E // _NSUB
_NB_E = _EDGES_PER_SUB // _EB
_EPS = 1e-5


def _mm0_body(x_ref, w_ref, out_ref):
    o = jnp.dot(x_ref[...], w_ref[...], preferred_element_type=jnp.float32)
    out_ref[0] = o[:, :_HD]
    out_ref[1] = o[:, _HD:]


def _mm0(x, w):
    return pl.pallas_call(
        _mm0_body,
        grid=(_NRB,),
        in_specs=[pl.BlockSpec((_RB, _D), lambda i: (i, 0)),
                  pl.BlockSpec((_D, _D), lambda i: (0, 0))],
        out_specs=pl.BlockSpec((2, _RB, _HD), lambda i: (0, i, 0)),
        out_shape=jax.ShapeDtypeStruct((2, _N, _HD), jnp.float32),
    )(x, w)


def _conv_body(hl_hbm, srcc_hbm, dst_hbm, ew_hbm, z_hbm, out_hbm,
               acc, gbuf, sbuf, dbuf, wbuf, sem):
    c = lax.axis_index("c")
    s = lax.axis_index("s")
    rbase = s * _ROWS_PER_SUB
    pltpu.sync_copy(z_hbm, acc.at[pl.ds(rbase, _ROWS_PER_SUB)])

    @pl.when(s == _NSUB - 1)
    def _():
        pltpu.sync_copy(z_hbm.at[pl.ds(0, _ROWS_TAIL)],
                        acc.at[pl.ds(_NSUB * _ROWS_PER_SUB, _ROWS_TAIL)])

    plsc.subcore_barrier()

    ebase = c * _E + s * _EDGES_PER_SUB

    def body(j, carry):
        eb = ebase + j * _EB
        db = s * _EDGES_PER_SUB + j * _EB
        pltpu.sync_copy(srcc_hbm.at[pl.ds(eb, _EB)], sbuf)
        pltpu.sync_copy(dst_hbm.at[pl.ds(db, _EB)], dbuf)
        pltpu.sync_copy(ew_hbm.at[pl.ds(db, _EB)], wbuf)
        pltpu.async_copy(hl_hbm.at[sbuf], gbuf, sem).wait()

        def scale(g, carry2):
            w16 = wbuf[pl.ds(g * 16, 16)]
            for u in range(16):
                e = g * 16 + u
                wv = jnp.full((16,), w16[u], jnp.float32)
                for q in range(_HD // 16):
                    gbuf[e, pl.ds(q * 16, 16)] = gbuf[e, pl.ds(q * 16, 16)] * wv
            return carry2

        lax.fori_loop(0, _EB // 16, scale, 0)
        pltpu.sync_copy(gbuf, acc.at[dbuf], add=True)
        return carry

    lax.fori_loop(0, _NB_E, body, 0)
    plsc.subcore_barrier()
    pltpu.sync_copy(acc.at[pl.ds(rbase, _ROWS_PER_SUB)],
                    out_hbm.at[pl.ds(c * _N + rbase, _ROWS_PER_SUB)])

    @pl.when(s == _NSUB - 1)
    def _():
        tbase = _NSUB * _ROWS_PER_SUB
        pltpu.sync_copy(acc.at[pl.ds(tbase, _ROWS_TAIL)],
                        out_hbm.at[pl.ds(c * _N + tbase, _ROWS_TAIL)])


@functools.cache
def _conv_sc():
    return pl.kernel(
        _conv_body,
        out_type=jax.ShapeDtypeStruct((2 * _N, _HD), jnp.float32),
        mesh=plsc.VectorSubcoreMesh(core_axis_name="c", subcore_axis_name="s",
                                    num_cores=2, num_subcores=_NSUB),
        scratch_types=[
            pltpu.VMEM_SHARED((_N, _HD), jnp.float32),
            pltpu.VMEM((_EB, _HD), jnp.float32),
            pltpu.VMEM((_EB,), jnp.int32),
            pltpu.VMEM((_EB,), jnp.int32),
            pltpu.VMEM((_EB,), jnp.float32),
            pltpu.SemaphoreType.DMA,
        ],
    )


def _conv(hl2, src_cat, dst, ew, zeros):
    out = _conv_sc()(hl2.reshape(2 * _N, _HD), src_cat, dst, ew, zeros)
    return out.reshape(2, _N, _HD)


def _stats_body(m_ref, out_ref):
    i = pl.program_id(0)

    @pl.when(i == 0)
    def _():
        out_ref[...] = jnp.zeros_like(out_ref)

    for j in range(2):
        mj = m_ref[j]
        s1 = jnp.sum(mj, axis=0, keepdims=True)
        s2 = jnp.sum(mj * mj, axis=0, keepdims=True)
        out_ref[j] = out_ref[j] + jnp.concatenate([s1, s2], axis=0)


def _stats(m):
    return pl.pallas_call(
        _stats_body,
        grid=(_NRB,),
        in_specs=[pl.BlockSpec((2, _RB, _HD), lambda i: (0, i, 0))],
        out_specs=pl.BlockSpec((2, 2, _HD), lambda i: (0, 0, 0)),
        out_shape=jax.ShapeDtypeStruct((2, 2, _HD), jnp.float32),
    )(m)


def _bn_block(m_ref, st_ref, g_ref, be_ref):
    hs = []
    for j in range(2):
        mj = m_ref[j]
        su = st_ref[j, 0:1, :]
        sq = st_ref[j, 1:2, :]
        mu = su / _N
        var = sq / _N - mu * mu
        inv = lax.rsqrt(var + _EPS) * g_ref[pl.ds(j, 1), :]
        hs.append(jnp.maximum((mj - mu) * inv + be_ref[pl.ds(j, 1), :], 0.0))
    return hs


def _bnmm_body(m_ref, st_ref, g_ref, be_ref, w_ref, h_ref, hl_ref):
    hs = _bn_block(m_ref, st_ref, g_ref, be_ref)
    h_ref[0] = hs[0]
    h_ref[1] = hs[1]
    o = jnp.dot(jnp.concatenate(hs, axis=1), w_ref[...],
                preferred_element_type=jnp.float32)
    hl_ref[0] = o[:, :_HD]
    hl_ref[1] = o[:, _HD:]


def _bnmm(m, st, g, be, w):
    return pl.pallas_call(
        _bnmm_body,
        grid=(_NRB,),
        in_specs=[pl.BlockSpec((2, _RB, _HD), lambda i: (0, i, 0)),
                  pl.BlockSpec((2, 2, _HD), lambda i: (0, 0, 0)),
                  pl.BlockSpec((2, _HD), lambda i: (0, 0)),
                  pl.BlockSpec((2, _HD), lambda i: (0, 0)),
                  pl.BlockSpec((_D, _D), lambda i: (0, 0))],
        out_specs=[pl.BlockSpec((2, _RB, _HD), lambda i: (0, i, 0)),
                   pl.BlockSpec((2, _RB, _HD), lambda i: (0, i, 0))],
        out_shape=[jax.ShapeDtypeStruct((2, _N, _HD), jnp.float32),
                   jax.ShapeDtypeStruct((2, _N, _HD), jnp.float32)],
    )(m, st, g, be, w)


def _pool_body(b_ref, x_ref, h1_ref, m1_ref, st_ref, g_ref, be_ref, s_ref):
    i = pl.program_id(0)

    @pl.when(i == 0)
    def _():
        s_ref[...] = jnp.zeros_like(s_ref)

    b = b_ref[0]
    gids = lax.broadcasted_iota(jnp.int32, (_G, _RB), 0)
    p = (b == gids).astype(jnp.float32)
    s_ref[0] = s_ref[0] + jnp.dot(p, x_ref[...],
                                  preferred_element_type=jnp.float32)
    h1 = jnp.concatenate([h1_ref[0], h1_ref[1]], axis=1)
    s_ref[1] = s_ref[1] + jnp.dot(p, h1, preferred_element_type=jnp.float32)
    h2 = jnp.concatenate(_bn_block(m1_ref, st_ref, g_ref, be_ref), axis=1)
    s_ref[2] = s_ref[2] + jnp.dot(p, h2, preferred_element_type=jnp.float32)
    ones = jnp.ones((_RB, _D), jnp.float32)
    s_ref[3] = s_ref[3] + jnp.dot(p, ones, preferred_element_type=jnp.float32)


def _pool(batch_r, x, h1, m1, st, g, be):
    return pl.pallas_call(
        _pool_body,
        grid=(_NRB,),
        in_specs=[pl.BlockSpec((1, 1, _RB), lambda i: (i, 0, 0)),
                  pl.BlockSpec((_RB, _D), lambda i: (i, 0)),
                  pl.BlockSpec((2, _RB, _HD), lambda i: (0, i, 0)),
                  pl.BlockSpec((2, _RB, _HD), lambda i: (0, i, 0)),
                  pl.BlockSpec((2, 2, _HD), lambda i: (0, 0, 0)),
                  pl.BlockSpec((2, _HD), lambda i: (0, 0)),
                  pl.BlockSpec((2, _HD), lambda i: (0, 0))],
        out_specs=pl.BlockSpec((4, _G, _D), lambda i: (0, 0, 0)),
        out_shape=jax.ShapeDtypeStruct((4, _G, _D), jnp.float32),
    )(batch_r, x, h1, m1, st, g, be)


def _readout_body(s_ref, wp_ref, bp_ref, out_ref):
    inv = 1.0 / jnp.maximum(s_ref[3, :, 0:1], 1.0)
    acc = jnp.broadcast_to(bp_ref[...], (_G, _DO))
    for i in range(3):
        acc = acc + jnp.dot(s_ref[i] * inv, wp_ref[i],
                            preferred_element_type=jnp.float32)
    out_ref[...] = jax.nn.sigmoid(acc)


def _readout(s, wp, bp):
    return pl.pallas_call(
        _readout_body,
        grid=(1,),
        in_specs=[pl.BlockSpec((4, _G, _D), lambda i: (0, 0, 0)),
                  pl.BlockSpec((3, _D, _DO), lambda i: (0, 0, 0)),
                  pl.BlockSpec((1, _DO), lambda i: (0, 0))],
        out_specs=pl.BlockSpec((_G, _DO), lambda i: (0, 0)),
        out_shape=jax.ShapeDtypeStruct((_G, _DO), jnp.float32),
    )(s, wp, bp)


def kernel(x, edge_index, edge_weight, batch,
           W0, b0, W1, b1, g0, be0, g1, be1,
           Wp0, bp0, Wp1, bp1, Wp2, bp2):
    src = edge_index[0]
    dst = edge_index[1]
    src_cat = jnp.concatenate([src, src + _N])
    zeros = jnp.zeros((_ROWS_PER_SUB, _HD), jnp.float32)
    batch_r = batch.reshape(_NRB, 1, _RB)
    g0r, be0r = g0.reshape(2, _HD), be0.reshape(2, _HD)
    g1r, be1r = g1.reshape(2, _HD), be1.reshape(2, _HD)
    wp = jnp.stack([Wp0, Wp1, Wp2])
    bp = (bp0 + bp1 + bp2).reshape(1, _DO)

    hl0 = _mm0(x, W0)
    m0 = _conv(hl0, src_cat, dst, edge_weight, zeros)
    st0 = _stats(m0)
    h1, hl1 = _bnmm(m0, st0, g0r, be0r, W1)
    m1 = _conv(hl1, src_cat, dst, edge_weight, zeros)
    st1 = _stats(m1)
    s = _pool(batch_r, x, h1, m1, st1, g1r, be1r)
    return _readout(s, wp, bp)


# speedup vs baseline: 3.6535x; 1.3220x over previous
import functools

import jax
import jax.numpy as jnp
from jax import lax
from jax.experimental import pallas as pl
from jax.experimental.pallas import tpu as pltpu
from jax.experimental.pallas import tpu_sc as plsc

_N = 10000
_E = 160000
_D = 256
_HD = 128
_DO = 128
_G = 64
_RB = 400
_NRB = _N // _RB
_EB = 128
_NSUB = 16
_ROWS_PER_SUB = 624
_ROWS_TAIL = _N - _NSUB * _ROWS_PER_SUB
_EDGES_PER_SUB = _E // _NSUB
_NB_E = 80
_EPT = _NB_E * _EB
_EPS = 1e-5


def _mm0_body(x_ref, w_ref, out_ref):
    o = jnp.dot(x_ref[...], w_ref[...], preferred_element_type=jnp.float32)
    out_ref[0] = o[:, :_HD]
    out_ref[1] = o[:, _HD:]


def _mm0(x, w):
    return pl.pallas_call(
        _mm0_body,
        grid=(_NRB,),
        in_specs=[pl.BlockSpec((_RB, _D), lambda i: (i, 0)),
                  pl.BlockSpec((_D, _D), lambda i: (0, 0))],
        out_specs=pl.BlockSpec((2, _RB, _HD), lambda i: (0, i, 0)),
        out_shape=jax.ShapeDtypeStruct((2, _N, _HD), jnp.float32),
    )(x, w)


def _scale(gb, wb):
    def grp(g, carry):
        w16 = wb[pl.ds(g * 16, 16)]
        for u in range(16):
            e = g * 16 + u
            wv = jnp.full((16,), w16[u], jnp.float32)
            for q in range(_HD // 16):
                gb[e, pl.ds(q * 16, 16)] = gb[e, pl.ds(q * 16, 16)] * wv
        return carry

    lax.fori_loop(0, _EB // 16, grp, 0)


def _conv_body(hl_hbm, src_hbm, dst_hbm, ew_hbm, z_hbm, out_hbm,
               acc, sidx, gb0, gb1, db0, db1, wb0, wb1,
               gs0, gs1, ss0, ss1, ds0, ds1, ws0, ws1):
    c = lax.axis_index("c")
    s = lax.axis_index("s")
    rbase = s * _ROWS_PER_SUB
    pltpu.sync_copy(z_hbm, acc.at[pl.ds(rbase, _ROWS_PER_SUB)])

    @pl.when(s == _NSUB - 1)
    def _():
        pltpu.sync_copy(z_hbm.at[pl.ds(0, _ROWS_TAIL)],
                        acc.at[pl.ds(_NSUB * _ROWS_PER_SUB, _ROWS_TAIL)])

    pltpu.sync_copy(src_hbm.at[pl.ds((c * _NSUB + s) * _NB_E, _NB_E)], sidx)
    plsc.subcore_barrier()

    ebase = s * _EPT

    def d_start(j, db, dsem, wb, wsem):
        pltpu.async_copy(dst_hbm.at[pl.ds(ebase + j * _EB, _EB)], db, dsem)
        pltpu.async_copy(ew_hbm.at[pl.ds(ebase + j * _EB, _EB)], wb, wsem)

    def d_wait(j, db, dsem, wb, wsem):
        pltpu.make_async_copy(dst_hbm.at[pl.ds(ebase + j * _EB, _EB)],
                              db, dsem).wait()
        pltpu.make_async_copy(ew_hbm.at[pl.ds(ebase + j * _EB, _EB)],
                              wb, wsem).wait()

    def g_start(j, gb, gsem):
        pltpu.async_copy(hl_hbm.at[sidx.at[j]], gb, gsem)

    def g_wait(j, gb, gsem):
        pltpu.make_async_copy(hl_hbm.at[sidx.at[j]], gb, gsem).wait()

    def sc_start(gb, db, ssem):
        pltpu.async_copy(gb, acc.at[db], ssem, add=True)

    def sc_wait(gb, db, ssem):
        pltpu.make_async_copy(gb, acc.at[db], ssem).wait()

    d_start(0, db0, ds0, wb0, ws0)
    g_start(0, gb0, gs0)

    def pair(k, carry):
        j0 = 2 * k
        j1 = j0 + 1
        g_wait(j0, gb0, gs0)

        @pl.when(k > 0)
        def _():
            sc_wait(gb1, db1, ss1)

        g_start(j1, gb1, gs1)
        d_start(j1, db1, ds1, wb1, ws1)
        d_wait(j0, db0, ds0, wb0, ws0)
        _scale(gb0, wb0)
        sc_start(gb0, db0, ss0)
        g_wait(j1, gb1, gs1)
        d_wait(j1, db1, ds1, wb1, ws1)
        _scale(gb1, wb1)
        sc_wait(gb0, db0, ss0)

        @pl.when(k < _NB_E // 2 - 1)
        def _():
            g_start(j0 + 2, gb0, gs0)
            d_start(j0 + 2, db0, ds0, wb0, ws0)

        sc_start(gb1, db1, ss1)
        return carry

    lax.fori_loop(0, _NB_E // 2, pair, 0)
    sc_wait(gb1, db1, ss1)
    plsc.subcore_barrier()
    pltpu.sync_copy(acc.at[pl.ds(rbase, _ROWS_PER_SUB)],
                    out_hbm.at[pl.ds(c * _N + rbase, _ROWS_PER_SUB)])

    @pl.when(s == _NSUB - 1)
    def _():
        tbase = _NSUB * _ROWS_PER_SUB
        pltpu.sync_copy(acc.at[pl.ds(tbase, _ROWS_TAIL)],
                        out_hbm.at[pl.ds(c * _N + tbase, _ROWS_TAIL)])


@functools.cache
def _conv_sc():
    return pl.kernel(
        _conv_body,
        out_type=jax.ShapeDtypeStruct((2 * _N, _HD), jnp.float32),
        mesh=plsc.VectorSubcoreMesh(core_axis_name="c", subcore_axis_name="s",
                                    num_cores=2, num_subcores=_NSUB),
        scratch_types=[
            pltpu.VMEM_SHARED((_N, _HD), jnp.float32),
            pltpu.VMEM((_NB_E, _EB), jnp.int32),
            pltpu.VMEM((_EB, _HD), jnp.float32),
            pltpu.VMEM((_EB, _HD), jnp.float32),
            pltpu.VMEM((_EB,), jnp.int32),
            pltpu.VMEM((_EB,), jnp.int32),
            pltpu.VMEM((_EB,), jnp.float32),
            pltpu.VMEM((_EB,), jnp.float32),
        ] + [pltpu.SemaphoreType.DMA] * 8,
    )


def _pad_edges(a):
    a = a.reshape(_NSUB, _EDGES_PER_SUB)
    return jnp.pad(a, ((0, 0), (0, _EPT - _EDGES_PER_SUB)))


def _conv(hl2, src_p, dst_p, ew_p, zeros):
    out = _conv_sc()(hl2.reshape(2 * _N, _HD), src_p, dst_p, ew_p, zeros)
    return out.reshape(2, _N, _HD)


def _stats_body(m_ref, out_ref):
    i = pl.program_id(0)

    @pl.when(i == 0)
    def _():
        out_ref[...] = jnp.zeros_like(out_ref)

    for j in range(2):
        mj = m_ref[j]
        s1 = jnp.sum(mj, axis=0, keepdims=True)
        s2 = jnp.sum(mj * mj, axis=0, keepdims=True)
        out_ref[j] = out_ref[j] + jnp.concatenate([s1, s2], axis=0)


def _stats(m):
    return pl.pallas_call(
        _stats_body,
        grid=(_NRB,),
        in_specs=[pl.BlockSpec((2, _RB, _HD), lambda i: (0, i, 0))],
        out_specs=pl.BlockSpec((2, 2, _HD), lambda i: (0, 0, 0)),
        out_shape=jax.ShapeDtypeStruct((2, 2, _HD), jnp.float32),
    )(m)


def _bn_block(m_ref, st_ref, g_ref, be_ref):
    hs = []
    for j in range(2):
        mj = m_ref[j]
        su = st_ref[j, 0:1, :]
        sq = st_ref[j, 1:2, :]
        mu = su / _N
        var = sq / _N - mu * mu
        inv = lax.rsqrt(var + _EPS) * g_ref[pl.ds(j, 1), :]
        hs.append(jnp.maximum((mj - mu) * inv + be_ref[pl.ds(j, 1), :], 0.0))
    return hs


def _bnmm_body(m_ref, st_ref, g_ref, be_ref, w_ref, h_ref, hl_ref):
    hs = _bn_block(m_ref, st_ref, g_ref, be_ref)
    h_ref[0] = hs[0]
    h_ref[1] = hs[1]
    o = jnp.dot(jnp.concatenate(hs, axis=1), w_ref[...],
                preferred_element_type=jnp.float32)
    hl_ref[0] = o[:, :_HD]
    hl_ref[1] = o[:, _HD:]


def _bnmm(m, st, g, be, w):
    return pl.pallas_call(
        _bnmm_body,
        grid=(_NRB,),
        in_specs=[pl.BlockSpec((2, _RB, _HD), lambda i: (0, i, 0)),
                  pl.BlockSpec((2, 2, _HD), lambda i: (0, 0, 0)),
                  pl.BlockSpec((2, _HD), lambda i: (0, 0)),
                  pl.BlockSpec((2, _HD), lambda i: (0, 0)),
                  pl.BlockSpec((_D, _D), lambda i: (0, 0))],
        out_specs=[pl.BlockSpec((2, _RB, _HD), lambda i: (0, i, 0)),
                   pl.BlockSpec((2, _RB, _HD), lambda i: (0, i, 0))],
        out_shape=[jax.ShapeDtypeStruct((2, _N, _HD), jnp.float32),
                   jax.ShapeDtypeStruct((2, _N, _HD), jnp.float32)],
    )(m, st, g, be, w)


def _pool_body(b_ref, x_ref, h1_ref, m1_ref, st_ref, g_ref, be_ref, s_ref):
    i = pl.program_id(0)

    @pl.when(i == 0)
    def _():
        s_ref[...] = jnp.zeros_like(s_ref)

    b = b_ref[0]
    gids = lax.broadcasted_iota(jnp.int32, (_G, _RB), 0)
    p = (b == gids).astype(jnp.float32)
    s_ref[0] = s_ref[0] + jnp.dot(p, x_ref[...],
                                  preferred_element_type=jnp.float32)
    h1 = jnp.concatenate([h1_ref[0], h1_ref[1]], axis=1)
    s_ref[1] = s_ref[1] + jnp.dot(p, h1, preferred_element_type=jnp.float32)
    h2 = jnp.concatenate(_bn_block(m1_ref, st_ref, g_ref, be_ref), axis=1)
    s_ref[2] = s_ref[2] + jnp.dot(p, h2, preferred_element_type=jnp.float32)
    ones = jnp.ones((_RB, _D), jnp.float32)
    s_ref[3] = s_ref[3] + jnp.dot(p, ones, preferred_element_type=jnp.float32)


def _pool(batch_r, x, h1, m1, st, g, be):
    return pl.pallas_call(
        _pool_body,
        grid=(_NRB,),
        in_specs=[pl.BlockSpec((1, 1, _RB), lambda i: (i, 0, 0)),
                  pl.BlockSpec((_RB, _D), lambda i: (i, 0)),
                  pl.BlockSpec((2, _RB, _HD), lambda i: (0, i, 0)),
                  pl.BlockSpec((2, _RB, _HD), lambda i: (0, i, 0)),
                  pl.BlockSpec((2, 2, _HD), lambda i: (0, 0, 0)),
                  pl.BlockSpec((2, _HD), lambda i: (0, 0)),
                  pl.BlockSpec((2, _HD), lambda i: (0, 0))],
        out_specs=pl.BlockSpec((4, _G, _D), lambda i: (0, 0, 0)),
        out_shape=jax.ShapeDtypeStruct((4, _G, _D), jnp.float32),
    )(batch_r, x, h1, m1, st, g, be)


def _readout_body(s_ref, wp_ref, bp_ref, out_ref):
    inv = 1.0 / jnp.maximum(s_ref[3, :, 0:1], 1.0)
    acc = jnp.broadcast_to(bp_ref[...], (_G, _DO))
    for i in range(3):
        acc = acc + jnp.dot(s_ref[i] * inv, wp_ref[i],
                            preferred_element_type=jnp.float32)
    out_ref[...] = jax.nn.sigmoid(acc)


def _readout(s, wp, bp):
    return pl.pallas_call(
        _readout_body,
        grid=(1,),
        in_specs=[pl.BlockSpec((4, _G, _D), lambda i: (0, 0, 0)),
                  pl.BlockSpec((3, _D, _DO), lambda i: (0, 0, 0)),
                  pl.BlockSpec((1, _DO), lambda i: (0, 0))],
        out_specs=pl.BlockSpec((_G, _DO), lambda i: (0, 0)),
        out_shape=jax.ShapeDtypeStruct((_G, _DO), jnp.float32),
    )(s, wp, bp)


def kernel(x, edge_index, edge_weight, batch,
           W0, b0, W1, b1, g0, be0, g1, be1,
           Wp0, bp0, Wp1, bp1, Wp2, bp2):
    src = edge_index[0]
    dst = edge_index[1]
    src_t = _pad_edges(src)
    src_p = jnp.stack([src_t, src_t + _N]).reshape(-1, _EB)
    dst_p = _pad_edges(dst).reshape(-1)
    ew_p = _pad_edges(edge_weight).reshape(-1)
    zeros = jnp.zeros((_ROWS_PER_SUB, _HD), jnp.float32)
    batch_r = batch.reshape(_NRB, 1, _RB)
    g0r, be0r = g0.reshape(2, _HD), be0.reshape(2, _HD)
    g1r, be1r = g1.reshape(2, _HD), be1.reshape(2, _HD)
    wp = jnp.stack([Wp0, Wp1, Wp2])
    bp = (bp0 + bp1 + bp2).reshape(1, _DO)

    hl0 = _mm0(x, W0)
    m0 = _conv(hl0, src_p, dst_p, ew_p, zeros)
    st0 = _stats(m0)
    h1, hl1 = _bnmm(m0, st0, g0r, be0r, W1)
    m1 = _conv(hl1, src_p, dst_p, ew_p, zeros)
    st1 = _stats(m1)
    s = _pool(batch_r, x, h1, m1, st1, g1r, be1r)
    return _readout(s, wp, bp)


# speedup vs baseline: 3.8485x; 1.0534x over previous
import functools

import jax
import jax.numpy as jnp
from jax import lax
from jax.experimental import pallas as pl
from jax.experimental.pallas import tpu as pltpu
from jax.experimental.pallas import tpu_sc as plsc

_N = 10000
_E = 160000
_D = 256
_HD = 128
_DO = 128
_G = 64
_RB = 400
_NRB = _N // _RB
_EB = 128
_NSUB = 16
_ROWS_PER_SUB = 624
_ROWS_TAIL = _N - _NSUB * _ROWS_PER_SUB
_EDGES_PER_SUB = _E // _NSUB
_NB_E = 80
_EPT = _NB_E * _EB
_EPS = 1e-5


def _mm0_body(x_ref, w_ref, out_ref):
    o = jnp.dot(x_ref[...], w_ref[...], preferred_element_type=jnp.float32)
    out_ref[0] = o[:, :_HD]
    out_ref[1] = o[:, _HD:]


def _mm0(x, w):
    return pl.pallas_call(
        _mm0_body,
        grid=(_NRB,),
        in_specs=[pl.BlockSpec((_RB, _D), lambda i: (i, 0)),
                  pl.BlockSpec((_D, _D), lambda i: (0, 0))],
        out_specs=pl.BlockSpec((2, _RB, _HD), lambda i: (0, i, 0)),
        out_shape=jax.ShapeDtypeStruct((2, _N, _HD), jnp.float32),
    )(x, w)


def _scale(gb, wb):
    def grp(g, carry):
        w16 = wb[pl.ds(g * 16, 16)]
        for u in range(16):
            e = g * 16 + u
            wv = jnp.full((16,), w16[u], jnp.float32)
            for q in range(_HD // 16):
                gb[e, pl.ds(q * 16, 16)] = gb[e, pl.ds(q * 16, 16)] * wv
        return carry

    lax.fori_loop(0, _EB // 16, grp, 0)


def _conv_body(hl_hbm, src_hbm, dst_hbm, ew_hbm, z_hbm, out_hbm,
               acc, sidx, gb0, gb1, db0, db1, wb0, wb1,
               gs0, gs1, ss0, ss1, ds0, ds1, ws0, ws1):
    c = lax.axis_index("c")
    s = lax.axis_index("s")
    rbase = s * _ROWS_PER_SUB
    pltpu.sync_copy(z_hbm, acc.at[pl.ds(rbase, _ROWS_PER_SUB)])

    @pl.when(s == _NSUB - 1)
    def _():
        pltpu.sync_copy(z_hbm.at[pl.ds(0, _ROWS_TAIL)],
                        acc.at[pl.ds(_NSUB * _ROWS_PER_SUB, _ROWS_TAIL)])

    pltpu.sync_copy(src_hbm.at[pl.ds((c * _NSUB + s) * _NB_E, _NB_E)], sidx)
    plsc.subcore_barrier()

    ebase = s * _EPT

    def d_start(j, db, dsem, wb, wsem):
        pltpu.async_copy(dst_hbm.at[pl.ds(ebase + j * _EB, _EB)], db, dsem)
        pltpu.async_copy(ew_hbm.at[pl.ds(ebase + j * _EB, _EB)], wb, wsem)

    def d_wait(j, db, dsem, wb, wsem):
        pltpu.make_async_copy(dst_hbm.at[pl.ds(ebase + j * _EB, _EB)],
                              db, dsem).wait()
        pltpu.make_async_copy(ew_hbm.at[pl.ds(ebase + j * _EB, _EB)],
                              wb, wsem).wait()

    def g_start(j, gb, gsem):
        pltpu.async_copy(hl_hbm.at[sidx.at[j]], gb, gsem)

    def g_wait(j, gb, gsem):
        pltpu.make_async_copy(hl_hbm.at[sidx.at[j]], gb, gsem).wait()

    def sc_start(gb, db, ssem):
        pltpu.async_copy(gb, acc.at[db], ssem, add=True)

    def sc_wait(gb, db, ssem):
        pltpu.make_async_copy(gb, acc.at[db], ssem).wait()

    d_start(0, db0, ds0, wb0, ws0)
    g_start(0, gb0, gs0)

    def pair(k, carry):
        j0 = 2 * k
        j1 = j0 + 1
        @pl.when(k > 0)
        def _():
            sc_wait(gb1, db1, ss1)

        g_start(j1, gb1, gs1)
        d_start(j1, db1, ds1, wb1, ws1)
        g_wait(j0, gb0, gs0)
        d_wait(j0, db0, ds0, wb0, ws0)
        _scale(gb0, wb0)
        sc_start(gb0, db0, ss0)
        sc_wait(gb0, db0, ss0)

        @pl.when(k < _NB_E // 2 - 1)
        def _():
            g_start(j0 + 2, gb0, gs0)
            d_start(j0 + 2, db0, ds0, wb0, ws0)

        g_wait(j1, gb1, gs1)
        d_wait(j1, db1, ds1, wb1, ws1)
        _scale(gb1, wb1)
        sc_start(gb1, db1, ss1)
        return carry

    lax.fori_loop(0, _NB_E // 2, pair, 0)
    sc_wait(gb1, db1, ss1)
    plsc.subcore_barrier()
    pltpu.sync_copy(acc.at[pl.ds(rbase, _ROWS_PER_SUB)],
                    out_hbm.at[pl.ds(c * _N + rbase, _ROWS_PER_SUB)])

    @pl.when(s == _NSUB - 1)
    def _():
        tbase = _NSUB * _ROWS_PER_SUB
        pltpu.sync_copy(acc.at[pl.ds(tbase, _ROWS_TAIL)],
                        out_hbm.at[pl.ds(c * _N + tbase, _ROWS_TAIL)])


@functools.cache
def _conv_sc():
    return pl.kernel(
        _conv_body,
        out_type=jax.ShapeDtypeStruct((2 * _N, _HD), jnp.float32),
        mesh=plsc.VectorSubcoreMesh(core_axis_name="c", subcore_axis_name="s",
                                    num_cores=2, num_subcores=_NSUB),
        scratch_types=[
            pltpu.VMEM_SHARED((_N, _HD), jnp.float32),
            pltpu.VMEM((_NB_E, _EB), jnp.int32),
            pltpu.VMEM((_EB, _HD), jnp.float32),
            pltpu.VMEM((_EB, _HD), jnp.float32),
            pltpu.VMEM((_EB,), jnp.int32),
            pltpu.VMEM((_EB,), jnp.int32),
            pltpu.VMEM((_EB,), jnp.float32),
            pltpu.VMEM((_EB,), jnp.float32),
        ] + [pltpu.SemaphoreType.DMA] * 8,
    )


def _pad_edges(a):
    a = a.reshape(_NSUB, _EDGES_PER_SUB)
    return jnp.pad(a, ((0, 0), (0, _EPT - _EDGES_PER_SUB)))


def _conv(hl2, src_p, dst_p, ew_p, zeros):
    out = _conv_sc()(hl2.reshape(2 * _N, _HD), src_p, dst_p, ew_p, zeros)
    return out.reshape(2, _N, _HD)


def _stats_body(m_ref, out_ref):
    i = pl.program_id(0)

    @pl.when(i == 0)
    def _():
        out_ref[...] = jnp.zeros_like(out_ref)

    for j in range(2):
        mj = m_ref[j]
        s1 = jnp.sum(mj, axis=0, keepdims=True)
        s2 = jnp.sum(mj * mj, axis=0, keepdims=True)
        out_ref[j] = out_ref[j] + jnp.concatenate([s1, s2], axis=0)


def _stats(m):
    return pl.pallas_call(
        _stats_body,
        grid=(_NRB,),
        in_specs=[pl.BlockSpec((2, _RB, _HD), lambda i: (0, i, 0))],
        out_specs=pl.BlockSpec((2, 2, _HD), lambda i: (0, 0, 0)),
        out_shape=jax.ShapeDtypeStruct((2, 2, _HD), jnp.float32),
    )(m)


def _bn_block(m_ref, st_ref, g_ref, be_ref):
    hs = []
    for j in range(2):
        mj = m_ref[j]
        su = st_ref[j, 0:1, :]
        sq = st_ref[j, 1:2, :]
        mu = su / _N
        var = sq / _N - mu * mu
        inv = lax.rsqrt(var + _EPS) * g_ref[pl.ds(j, 1), :]
        hs.append(jnp.maximum((mj - mu) * inv + be_ref[pl.ds(j, 1), :], 0.0))
    return hs


def _bnmm_body(m_ref, st_ref, g_ref, be_ref, w_ref, h_ref, hl_ref):
    hs = _bn_block(m_ref, st_ref, g_ref, be_ref)
    h_ref[0] = hs[0]
    h_ref[1] = hs[1]
    o = jnp.dot(jnp.concatenate(hs, axis=1), w_ref[...],
                preferred_element_type=jnp.float32)
    hl_ref[0] = o[:, :_HD]
    hl_ref[1] = o[:, _HD:]


def _bnmm(m, st, g, be, w):
    return pl.pallas_call(
        _bnmm_body,
        grid=(_NRB,),
        in_specs=[pl.BlockSpec((2, _RB, _HD), lambda i: (0, i, 0)),
                  pl.BlockSpec((2, 2, _HD), lambda i: (0, 0, 0)),
                  pl.BlockSpec((2, _HD), lambda i: (0, 0)),
                  pl.BlockSpec((2, _HD), lambda i: (0, 0)),
                  pl.BlockSpec((_D, _D), lambda i: (0, 0))],
        out_specs=[pl.BlockSpec((2, _RB, _HD), lambda i: (0, i, 0)),
                   pl.BlockSpec((2, _RB, _HD), lambda i: (0, i, 0))],
        out_shape=[jax.ShapeDtypeStruct((2, _N, _HD), jnp.float32),
                   jax.ShapeDtypeStruct((2, _N, _HD), jnp.float32)],
    )(m, st, g, be, w)


def _pool_body(b_ref, x_ref, h1_ref, m1_ref, st_ref, g_ref, be_ref, s_ref):
    i = pl.program_id(0)

    @pl.when(i == 0)
    def _():
        s_ref[...] = jnp.zeros_like(s_ref)

    b = b_ref[0]
    gids = lax.broadcasted_iota(jnp.int32, (_G, _RB), 0)
    p = (b == gids).astype(jnp.float32)
    s_ref[0] = s_ref[0] + jnp.dot(p, x_ref[...],
                                  preferred_element_type=jnp.float32)
    h1 = jnp.concatenate([h1_ref[0], h1_ref[1]], axis=1)
    s_ref[1] = s_ref[1] + jnp.dot(p, h1, preferred_element_type=jnp.float32)
    h2 = jnp.concatenate(_bn_block(m1_ref, st_ref, g_ref, be_ref), axis=1)
    s_ref[2] = s_ref[2] + jnp.dot(p, h2, preferred_element_type=jnp.float32)
    ones = jnp.ones((_RB, _D), jnp.float32)
    s_ref[3] = s_ref[3] + jnp.dot(p, ones, preferred_element_type=jnp.float32)


def _pool(batch_r, x, h1, m1, st, g, be):
    return pl.pallas_call(
        _pool_body,
        grid=(_NRB,),
        in_specs=[pl.BlockSpec((1, 1, _RB), lambda i: (i, 0, 0)),
                  pl.BlockSpec((_RB, _D), lambda i: (i, 0)),
                  pl.BlockSpec((2, _RB, _HD), lambda i: (0, i, 0)),
                  pl.BlockSpec((2, _RB, _HD), lambda i: (0, i, 0)),
                  pl.BlockSpec((2, 2, _HD), lambda i: (0, 0, 0)),
                  pl.BlockSpec((2, _HD), lambda i: (0, 0)),
                  pl.BlockSpec((2, _HD), lambda i: (0, 0))],
        out_specs=pl.BlockSpec((4, _G, _D), lambda i: (0, 0, 0)),
        out_shape=jax.ShapeDtypeStruct((4, _G, _D), jnp.float32),
    )(batch_r, x, h1, m1, st, g, be)


def _readout_body(s_ref, wp_ref, bp_ref, out_ref):
    inv = 1.0 / jnp.maximum(s_ref[3, :, 0:1], 1.0)
    acc = jnp.broadcast_to(bp_ref[...], (_G, _DO))
    for i in range(3):
        acc = acc + jnp.dot(s_ref[i] * inv, wp_ref[i],
                            preferred_element_type=jnp.float32)
    out_ref[...] = jax.nn.sigmoid(acc)


def _readout(s, wp, bp):
    return pl.pallas_call(
        _readout_body,
        grid=(1,),
        in_specs=[pl.BlockSpec((4, _G, _D), lambda i: (0, 0, 0)),
                  pl.BlockSpec((3, _D, _DO), lambda i: (0, 0, 0)),
                  pl.BlockSpec((1, _DO), lambda i: (0, 0))],
        out_specs=pl.BlockSpec((_G, _DO), lambda i: (0, 0)),
        out_shape=jax.ShapeDtypeStruct((_G, _DO), jnp.float32),
    )(s, wp, bp)


def kernel(x, edge_index, edge_weight, batch,
           W0, b0, W1, b1, g0, be0, g1, be1,
           Wp0, bp0, Wp1, bp1, Wp2, bp2):
    src = edge_index[0]
    dst = edge_index[1]
    src_t = _pad_edges(src)
    src_p = jnp.stack([src_t, src_t + _N]).reshape(-1, _EB)
    dst_p = _pad_edges(dst).reshape(-1)
    ew_p = _pad_edges(edge_weight).reshape(-1)
    zeros = jnp.zeros((_ROWS_PER_SUB, _HD), jnp.float32)
    batch_r = batch.reshape(_NRB, 1, _RB)
    g0r, be0r = g0.reshape(2, _HD), be0.reshape(2, _HD)
    g1r, be1r = g1.reshape(2, _HD), be1.reshape(2, _HD)
    wp = jnp.stack([Wp0, Wp1, Wp2])
    bp = (bp0 + bp1 + bp2).reshape(1, _DO)

    hl0 = _mm0(x, W0)
    m0 = _conv(hl0, src_p, dst_p, ew_p, zeros)
    st0 = _stats(m0)
    h1, hl1 = _bnmm(m0, st0, g0r, be0r, W1)
    m1 = _conv(hl1, src_p, dst_p, ew_p, zeros)
    st1 = _stats(m1)
    s = _pool(batch_r, x, h1, m1, st1, g1r, be1r)
    return _readout(s, wp, bp)
